# initial kernel scaffold (unmeasured)
import jax
import jax.numpy as jnp
from jax import lax
from jax.experimental import pallas as pl
from jax.experimental.pallas import tpu as pltpu

N_DEV = 4


def kernel(x, w_mat):
    m, k_per = x.shape
    _, n = w_mat.shape
    ch = m // N_DEV
    n_hops = 2 * (N_DEV - 1)

    def body(x_ref, w_ref, out_ref, send_buf, recv_buf, send_sems, recv_sems):
        my = lax.axis_index("i")
        right = lax.rem(my + 1, N_DEV)
        left = lax.rem(my + N_DEV - 1, N_DEV)

        barrier_sem = pltpu.get_barrier_semaphore()
        for nbr in (left, right):
            pl.semaphore_signal(
                barrier_sem, inc=1,
                device_id=(nbr,), device_id_type=pl.DeviceIdType.MESH,
            )
        pl.semaphore_wait(barrier_sem, 2)

        out_ref[:, :] = jnp.dot(
            x_ref[:, :].astype(jnp.bfloat16),
            w_ref[:, :].astype(jnp.bfloat16),
            preferred_element_type=jnp.float32,
        )

        def hop(h, c_send):
            send_buf[:, :] = out_ref[pl.ds(c_send * ch, ch), :].astype(jnp.bfloat16)
            rdma = pltpu.make_async_remote_copy(
                src_ref=send_buf,
                dst_ref=recv_buf.at[h],
                send_sem=send_sems.at[h],
                recv_sem=recv_sems.at[h],
                device_id=(right,),
                device_id_type=pl.DeviceIdType.MESH,
            )
            rdma.start()
            rdma.wait()

        for h in range(N_DEV - 1):
            hop(h, lax.rem(my - h + N_DEV, N_DEV))
            c_recv = lax.rem(my - h - 1 + N_DEV, N_DEV)
            sl = pl.ds(c_recv * ch, ch)
            out_ref[sl, :] = out_ref[sl, :] + recv_buf[h].astype(jnp.float32)

        own = pl.ds(lax.rem(my + 1, N_DEV) * ch, ch)
        y = out_ref[own, :]
        c = 0.7978845608028654
        out_ref[own, :] = 0.5 * y * (1.0 + jnp.tanh(c * (y + 0.044715 * y * y * y)))

        for a in range(N_DEV - 1):
            hop(N_DEV - 1 + a, lax.rem(my + 1 - a + N_DEV, N_DEV))
            c_recv = lax.rem(my - a + N_DEV, N_DEV)
            out_ref[pl.ds(c_recv * ch, ch), :] = recv_buf[N_DEV - 1 + a].astype(
                jnp.float32
            )

    return pl.pallas_call(
        body,
        out_shape=jax.ShapeDtypeStruct((m, n), jnp.float32),
        in_specs=[
            pl.BlockSpec(memory_space=pltpu.VMEM),
            pl.BlockSpec(memory_space=pltpu.VMEM),
        ],
        out_specs=pl.BlockSpec(memory_space=pltpu.VMEM),
        scratch_shapes=[
            pltpu.VMEM((ch, n), jnp.bfloat16),
            pltpu.VMEM((n_hops, ch, n), jnp.bfloat16),
            pltpu.SemaphoreType.DMA((n_hops,)),
            pltpu.SemaphoreType.DMA((n_hops,)),
        ],
        compiler_params=pltpu.CompilerParams(collective_id=0),
    )(x, w_mat)


# baseline (device time: 96364 ns/iter reference)
import jax
import jax.numpy as jnp
from jax import lax
from jax.experimental import pallas as pl
from jax.experimental.pallas import tpu as pltpu

N_DEV = 4
GELU_C = 0.7978845608028654


def _gelu(y):
    return 0.5 * y * (1.0 + jnp.tanh(GELU_C * (y + 0.044715 * y * y * y)))


def kernel(x, w_mat):
    m, k_per = x.shape
    _, n = w_mat.shape
    ch = m // N_DEV
    qn = n // 4
    n_hops = 2 * (N_DEV - 1)

    def body(x_ref, w_ref, out_ref, xb, wb, sbuf,
             rbuf_r, rbuf_l, ssem_r, ssem_l, rsem_r, rsem_l):
        my = lax.axis_index("i")
        right = lax.rem(my + 1, N_DEV)
        left = lax.rem(my + N_DEV - 1, N_DEV)

        barrier_sem = pltpu.get_barrier_semaphore()
        for nbr in (left, right):
            pl.semaphore_signal(
                barrier_sem, inc=1,
                device_id=(nbr,), device_id_type=pl.DeviceIdType.MESH,
            )
        pl.semaphore_wait(barrier_sem, 2)

        def rows(c):
            return pl.ds(c * ch, ch)

        def cols(d, s):
            lo = (d * 2 + s) * qn
            return slice(lo, lo + qn)

        def send(d, s, h, src):
            rbuf, ssem, rsem, peer = (
                (rbuf_r, ssem_r, rsem_r, right) if d == 0
                else (rbuf_l, ssem_l, rsem_l, left)
            )
            rdma = pltpu.make_async_remote_copy(
                src_ref=src,
                dst_ref=rbuf.at[h, s],
                send_sem=ssem.at[h, s],
                recv_sem=rsem.at[h, s],
                device_id=(peer,),
                device_id_type=pl.DeviceIdType.MESH,
            )
            rdma.start()
            return rdma

        def rs_recv_chunk(d, h):
            return lax.rem(my - h - 1 + N_DEV, N_DEV) if d == 0 \
                else lax.rem(my + h + 1, N_DEV)

        def ag_recv_chunk(d, a):
            return lax.rem(my - a + N_DEV, N_DEV) if d == 0 \
                else lax.rem(my + a, N_DEV)

        xb[:, :] = x_ref[:, :].astype(jnp.bfloat16)
        wb[:, :] = w_ref[:, :].astype(jnp.bfloat16)

        def gemm_chunk(c):
            out_ref[rows(c), :] = jnp.dot(
                xb[rows(c), :], wb[:, :], preferred_element_type=jnp.float32
            )

        gemm_chunk(my)
        rd = {}
        for s in (0, 1):
            for d in (0, 1):
                sbuf[d, s, :, :] = out_ref[rows(my), cols(d, s)].astype(
                    jnp.bfloat16
                )
                rd[d, s] = send(d, s, 0, sbuf.at[d, s])
        gemm_chunk(lax.rem(my + 1, N_DEV))
        gemm_chunk(lax.rem(my + N_DEV - 1, N_DEV))
        gemm_chunk(lax.rem(my + 2, N_DEV))

        for h in range(N_DEV - 1):
            for s in (0, 1):
                for d in (0, 1):
                    rd[d, s].wait()
                    c = rs_recv_chunk(d, h)
                    rbuf = rbuf_r if d == 0 else rbuf_l
                    acc = out_ref[rows(c), cols(d, s)] + rbuf[
                        h, s, :, :
                    ].astype(jnp.float32)
                    if h < N_DEV - 2:
                        sbuf[d, s, :, :] = acc.astype(jnp.bfloat16)
                        rd[d, s] = send(d, s, h + 1, sbuf.at[d, s])
                    else:
                        g = _gelu(acc)
                        out_ref[rows(c), cols(d, s)] = g
                        sbuf[d, s, :, :] = g.astype(jnp.bfloat16)
                        rd[d, s] = send(d, s, N_DEV - 1, sbuf.at[d, s])

        for a in range(N_DEV - 1):
            h = N_DEV - 1 + a
            for s in (0, 1):
                for d in (0, 1):
                    rd[d, s].wait()
                    rbuf = rbuf_r if d == 0 else rbuf_l
                    if a < N_DEV - 2:
                        rd[d, s] = send(d, s, h + 1, rbuf.at[h, s])
                    c = ag_recv_chunk(d, a)
                    out_ref[rows(c), cols(d, s)] = rbuf[h, s, :, :].astype(
                        jnp.float32
                    )

    return pl.pallas_call(
        body,
        out_shape=jax.ShapeDtypeStruct((m, n), jnp.float32),
        in_specs=[
            pl.BlockSpec(memory_space=pltpu.VMEM),
            pl.BlockSpec(memory_space=pltpu.VMEM),
        ],
        out_specs=pl.BlockSpec(memory_space=pltpu.VMEM),
        scratch_shapes=[
            pltpu.VMEM((m, k_per), jnp.bfloat16),
            pltpu.VMEM((k_per, n), jnp.bfloat16),
            pltpu.VMEM((2, 2, ch, qn), jnp.bfloat16),
            pltpu.VMEM((n_hops, 2, ch, qn), jnp.bfloat16),
            pltpu.VMEM((n_hops, 2, ch, qn), jnp.bfloat16),
            pltpu.SemaphoreType.DMA((n_hops, 2)),
            pltpu.SemaphoreType.DMA((n_hops, 2)),
            pltpu.SemaphoreType.DMA((n_hops, 2)),
            pltpu.SemaphoreType.DMA((n_hops, 2)),
        ],
        compiler_params=pltpu.CompilerParams(
            collective_id=0, vmem_limit_bytes=100 * 1024 * 1024
        ),
    )(x, w_mat)
